# baseline (speedup 1.0000x reference)
import jax
import jax.numpy as jnp
from jax.experimental import pallas as pl
from jax.experimental.pallas import tpu as pltpu

TM = 1024
KC = 1024


def _router_block(x_ref, wt_ref, b_ref, out_ref, *, nk):
    k = pl.program_id(1)

    @pl.when(k == 0)
    def _init():
        out_ref[...] = jnp.broadcast_to(b_ref[...], out_ref.shape)

    kc = x_ref.shape[1]
    out_ref[...] += jnp.dot(
        x_ref[...], wt_ref[pl.ds(k * kc, kc), :],
        preferred_element_type=jnp.float32)

    @pl.when(k == nk - 1)
    def _softmax():
        logits = out_ref[...]
        m = jnp.max(logits, axis=-1, keepdims=True)
        e = jnp.exp(logits - m)
        out_ref[...] = e / jnp.sum(e, axis=-1, keepdims=True)


def kernel(x, W, b):
    tokens, d_model = x.shape
    num_experts = W.shape[0]
    wt = W.T
    b2 = b.reshape(1, num_experts)
    nk = d_model // KC
    grid = (tokens // TM, nk)
    import functools
    return pl.pallas_call(
        functools.partial(_router_block, nk=nk),
        grid=grid,
        in_specs=[
            pl.BlockSpec((TM, KC), lambda i, k: (i, k)),
            pl.BlockSpec((d_model, num_experts), lambda i, k: (0, 0)),
            pl.BlockSpec((1, num_experts), lambda i, k: (0, 0)),
        ],
        out_specs=pl.BlockSpec((TM, num_experts), lambda i, k: (i, 0)),
        out_shape=jax.ShapeDtypeStruct((tokens, num_experts), jnp.float32),
        compiler_params=pltpu.CompilerParams(
            dimension_semantics=("parallel", "arbitrary"),
        ),
    )(x, wt, b2)


# speedup vs baseline: 1.2714x; 1.2714x over previous
import functools

import jax
import jax.numpy as jnp
from jax.experimental import pallas as pl
from jax.experimental.pallas import tpu as pltpu

TM = 256
NBUF = 8


def _router_block(x_hbm, wt_ref, b_ref, out_ref, xbuf, sem):
    i = pl.program_id(0)
    nblk = pl.num_programs(0)

    def copy_for(blk, slot):
        return pltpu.make_async_copy(
            x_hbm.at[pl.ds(blk * TM, TM), :], xbuf.at[slot], sem.at[slot])

    @pl.when(i == 0)
    def _prologue():
        for d in range(NBUF - 1):
            copy_for(d, d).start()

    nxt = i + NBUF - 1

    @pl.when(nxt < nblk)
    def _issue_ahead():
        copy_for(nxt, jax.lax.rem(nxt, NBUF)).start()

    slot = jax.lax.rem(i, NBUF)
    copy_for(i, slot).wait()

    logits = jnp.dot(xbuf[slot], wt_ref[...],
                     preferred_element_type=jnp.float32)
    logits = logits + b_ref[...]
    m = jnp.max(logits, axis=-1, keepdims=True)
    e = jnp.exp(logits - m)
    out_ref[...] = e / jnp.sum(e, axis=-1, keepdims=True)


def kernel(x, W, b):
    tokens, d_model = x.shape
    num_experts = W.shape[0]
    wt = W.T
    b2 = b.reshape(1, num_experts)
    grid = (tokens // TM,)
    return pl.pallas_call(
        _router_block,
        grid=grid,
        in_specs=[
            pl.BlockSpec(memory_space=pltpu.MemorySpace.HBM),
            pl.BlockSpec((d_model, num_experts), lambda i: (0, 0)),
            pl.BlockSpec((1, num_experts), lambda i: (0, 0)),
        ],
        out_specs=pl.BlockSpec((TM, num_experts), lambda i: (i, 0)),
        out_shape=jax.ShapeDtypeStruct((tokens, num_experts), jnp.float32),
        scratch_shapes=[
            pltpu.VMEM((NBUF, TM, d_model), jnp.float32),
            pltpu.SemaphoreType.DMA((NBUF,)),
        ],
        compiler_params=pltpu.CompilerParams(
            dimension_semantics=("arbitrary",),
        ),
    )(x, wt, b2)
